# baseline (speedup 1.0000x reference)
import jax
import jax.numpy as jnp
from jax import lax
from jax.experimental import pallas as pl
from jax.experimental.pallas import tpu as pltpu
from jax.experimental.pallas import tpu_sc as plsc

_B = 16384
_D = 128
_V = 16

_INFO = plsc.get_sparse_core_info()
_NC, _NS, _L = _INFO.num_cores, _INFO.num_subcores, _INFO.num_lanes
_NW = _NC * _NS
_BPW = _B // _NW
_NBLK = _BPW // _L


def _body(sr_hbm, srates_hbm, table_hbm, out_hbm, idx_v, tab_v, rows_v, sem):
    wid = lax.axis_index("s") * _NC + lax.axis_index("c")
    base = wid * _BPW
    pltpu.async_copy(sr_hbm.at[pl.ds(base, _BPW)], idx_v, sem).wait()
    pltpu.sync_copy(table_hbm, tab_v)

    rowpos = lax.iota(jnp.int32, _L) * _D

    def blk(i, carry):
        idx16 = idx_v[pl.ds(i * _L, _L)]
        fbase = idx16 * _D
        obase = rowpos + i * (_L * _D)
        for c in range(_D):
            v = plsc.load_gather(tab_v, [fbase + c])
            plsc.store_scatter(rows_v, [obase + c], v)
        return carry

    lax.fori_loop(0, _NBLK, blk, 0)
    pltpu.sync_copy(rows_v, out_hbm.at[pl.ds(base * _D, _BPW * _D)])


def kernel(sr_values, sample_rates, embedding_table):
    sr = sr_values.astype(jnp.int32)
    srt = sample_rates.astype(jnp.int32)
    tab = embedding_table.astype(jnp.float32).reshape(_V * _D)
    mesh = plsc.VectorSubcoreMesh(core_axis_name="c", subcore_axis_name="s")
    f = pl.kernel(
        _body,
        mesh=mesh,
        out_type=jax.ShapeDtypeStruct((_B * _D,), jnp.float32),
        scratch_types=[
            pltpu.VMEM((_BPW,), jnp.int32),
            pltpu.VMEM((_V * _D,), jnp.float32),
            pltpu.VMEM((_BPW * _D,), jnp.float32),
            pltpu.SemaphoreType.DMA,
        ],
        compiler_params=pltpu.CompilerParams(needs_layout_passes=False),
    )
    return f(sr, srt, tab).reshape(_B, _D)


# speedup vs baseline: 2.5156x; 2.5156x over previous
import jax
import jax.numpy as jnp
from jax import lax
from jax.experimental import pallas as pl
from jax.experimental.pallas import tpu as pltpu
from jax.experimental.pallas import tpu_sc as plsc

_B = 16384
_D = 128
_V = 16

_INFO = plsc.get_sparse_core_info()
_NC, _NS, _L = _INFO.num_cores, _INFO.num_subcores, _INFO.num_lanes
_NW = _NC * _NS
_BPW = _B // _NW
_NBLK = _BPW // _L


def _body(sr_hbm, srates_hbm, table_hbm, out_hbm, idx_v, tab_v, rows_v, sem):
    wid = lax.axis_index("s") * _NC + lax.axis_index("c")
    base = wid * _BPW
    pltpu.async_copy(sr_hbm.at[pl.ds(base, _BPW)], idx_v, sem).wait()
    pltpu.sync_copy(table_hbm, tab_v)

    def blk(i, carry):
        idx16 = idx_v[pl.ds(i * _L, _L)] * _D
        for j in range(_L):
            tb = idx16[j]
            ob = (i * _L + j) * _D
            for c in range(0, _D, _L):
                rows_v[pl.ds(ob + c, _L)] = tab_v[pl.ds(tb + c, _L)]
        return carry

    lax.fori_loop(0, _NBLK, blk, 0)
    pltpu.sync_copy(rows_v, out_hbm.at[pl.ds(base * _D, _BPW * _D)])


def kernel(sr_values, sample_rates, embedding_table):
    sr = sr_values.astype(jnp.int32)
    srt = sample_rates.astype(jnp.int32)
    tab = embedding_table.astype(jnp.float32).reshape(_V * _D)
    mesh = plsc.VectorSubcoreMesh(core_axis_name="c", subcore_axis_name="s")
    f = pl.kernel(
        _body,
        mesh=mesh,
        out_type=jax.ShapeDtypeStruct((_B * _D,), jnp.float32),
        scratch_types=[
            pltpu.VMEM((_BPW,), jnp.int32),
            pltpu.VMEM((_V * _D,), jnp.float32),
            pltpu.VMEM((_BPW * _D,), jnp.float32),
            pltpu.SemaphoreType.DMA,
        ],
        compiler_params=pltpu.CompilerParams(needs_layout_passes=False),
    )
    return f(sr, srt, tab).reshape(_B, _D)
